# Initial kernel scaffold; baseline (speedup 1.0000x reference)
import jax
import jax.numpy as jnp
from jax.experimental import pallas as pl


def kernel(x, Wg, bg, W1, b1, W2, b2):
    raise NotImplementedError("write your pallas kernel here")


# speedup vs baseline: 1.4262x; 1.4262x over previous
import jax
import jax.numpy as jnp
from jax.experimental import pallas as pl
from jax.experimental.pallas import tpu as pltpu

T = 2048
D = 1024
H = 2048
C = 1024
E = 8
K = 2
TB = 256
NT = T // TB


def _moe_dense_kernel(x_ref, Wg_ref, bg_ref, W1_ref, b1_ref, W2_ref, b2_ref,
                      out_ref, gates_ref, aux_ref,
                      acc_s, gates_s, w0_s, w1_s, i0_s, i1_s, fp_s):
    e = pl.program_id(0)
    t = pl.program_id(1)
    tok = pl.ds(t * TB, TB)

    @pl.when(e == 0)
    def _router():
        x = x_ref[...]
        logits = jax.lax.dot_general(
            x, Wg_ref[...], (((1,), (0,)), ((), ())),
            preferred_element_type=jnp.float32,
        ) + bg_ref[...]
        m = jnp.max(logits, axis=-1, keepdims=True)
        ex = jnp.exp(logits - m)
        gates = ex / jnp.sum(ex, axis=-1, keepdims=True)
        gates_s[tok, :] = gates

        cols = jax.lax.broadcasted_iota(jnp.int32, (TB, E), 1)
        m0 = jnp.max(gates, axis=-1, keepdims=True)
        i0 = jnp.min(jnp.where(gates == m0, cols, E), axis=-1, keepdims=True)
        g_masked = jnp.where(cols == i0, -jnp.inf, gates)
        m1 = jnp.max(g_masked, axis=-1, keepdims=True)
        i1 = jnp.min(jnp.where(g_masked == m1, cols, E), axis=-1, keepdims=True)
        denom = m0 + m1 + 1e-8
        w0_s[tok, :] = m0 / denom
        w1_s[tok, :] = m1 / denom
        i0_s[tok, :] = i0
        i1_s[tok, :] = i1

        mask = jnp.logical_or(cols == i0, cols == i1).astype(jnp.float32)
        fsum = jnp.sum(mask, axis=0, keepdims=True)
        psum = jnp.sum(gates, axis=0, keepdims=True)

        @pl.when(t == 0)
        def _():
            fp_s[0:1, :] = fsum
            fp_s[1:2, :] = psum

        @pl.when(t > 0)
        def _():
            fp_s[0:1, :] += fsum
            fp_s[1:2, :] += psum

    x = x_ref[...]
    h = jax.lax.dot_general(
        x, W1_ref[...], (((1,), (0,)), ((), ())),
        preferred_element_type=jnp.float32,
    ) + b1_ref[...]
    h = jnp.maximum(h, 0.0)
    y = jax.lax.dot_general(
        h, W2_ref[...], (((1,), (0,)), ((), ())),
        preferred_element_type=jnp.float32,
    ) + b2_ref[...]

    scale = (jnp.where(i0_s[tok, :] == e, w0_s[tok, :], 0.0)
             + jnp.where(i1_s[tok, :] == e, w1_s[tok, :], 0.0))

    @pl.when(e == 0)
    def _init():
        acc_s[tok, :] = scale * y

    @pl.when(e > 0)
    def _acc():
        acc_s[tok, :] += scale * y

    @pl.when(e == E - 1)
    def _out():
        out_ref[...] = acc_s[tok, :]
        gates_ref[...] = gates_s[tok, :]

    @pl.when(jnp.logical_and(e == E - 1, t == NT - 1))
    def _fin():
        aux_ref[0, :] = (jnp.sum(fp_s[0:1, :] * fp_s[1:2, :],
                                 axis=1, keepdims=True)
                         * (E / (T * T)))[0, :]


def kernel(x, Wg, bg, W1, b1, W2, b2):
    out, gates, aux = pl.pallas_call(
        _moe_dense_kernel,
        grid=(E, NT),
        in_specs=[
            pl.BlockSpec((TB, D), lambda e, t: (t, 0)),
            pl.BlockSpec((D, E), lambda e, t: (0, 0)),
            pl.BlockSpec((1, E), lambda e, t: (0, 0)),
            pl.BlockSpec((None, D, H), lambda e, t: (e, 0, 0)),
            pl.BlockSpec((None, 1, H), lambda e, t: (e, 0, 0)),
            pl.BlockSpec((None, H, C), lambda e, t: (e, 0, 0)),
            pl.BlockSpec((None, 1, C), lambda e, t: (e, 0, 0)),
        ],
        out_specs=[
            pl.BlockSpec((TB, C), lambda e, t: (t, 0)),
            pl.BlockSpec((TB, E), lambda e, t: (t, 0)),
            pl.BlockSpec((1, 1), lambda e, t: (0, 0)),
        ],
        out_shape=[
            jax.ShapeDtypeStruct((T, C), jnp.float32),
            jax.ShapeDtypeStruct((T, E), jnp.float32),
            jax.ShapeDtypeStruct((1, 1), jnp.float32),
        ],
        scratch_shapes=[
            pltpu.VMEM((T, C), jnp.float32),
            pltpu.VMEM((T, E), jnp.float32),
            pltpu.VMEM((T, 1), jnp.float32),
            pltpu.VMEM((T, 1), jnp.float32),
            pltpu.VMEM((T, 1), jnp.int32),
            pltpu.VMEM((T, 1), jnp.int32),
            pltpu.VMEM((2, E), jnp.float32),
        ],
    )(x, Wg, bg[None, :], W1, b1[:, None, :], W2, b2[:, None, :])
    return out, aux[0, 0], gates
